# baseline (speedup 1.0000x reference)
import functools
import numpy as np
import jax
import jax.numpy as jnp
from jax.experimental import pallas as pl

_A = 9
_STRIDE = 16
_SCALES = (64.0, 128.0, 256.0)
_RATIOS = (0.5, 1.0, 2.0)


def _rpn_kernel(x_ref, wt_ref, bc_ref, wh_ref, bh_ref, obj_ref, bb_ref,
                *, c, h, w):
    hw = h * w
    x = x_ref[0]

    pos = jax.lax.broadcasted_iota(jnp.int32, (1, hw), 1)
    col = pos % w
    ones = jnp.ones((1, hw), jnp.float32)
    wmask = [ (col != 0).astype(jnp.float32),
              ones,
              (col != w - 1).astype(jnp.float32) ]
    hmask = [ (pos >= w).astype(jnp.float32),
              ones,
              (pos < hw - w).astype(jnp.float32) ]

    acc = jnp.zeros((c, hw), dtype=jnp.float32)
    for k in range(9):
        i, j = k // 3, k % 3
        s = (i - 1) * w + (j - 1)
        xs = x if s == 0 else jnp.roll(x, -s, axis=1)
        m = None
        if i != 1:
            m = hmask[i]
        if j != 1:
            m = wmask[j] if m is None else m * wmask[j]
        xs = xs if m is None else xs * m
        acc = acc + jnp.dot(wt_ref[k], xs, preferred_element_type=jnp.float32)
    y = jnp.maximum(acc + bc_ref[...], 0.0)
    head = jnp.dot(wh_ref[...], y, preferred_element_type=jnp.float32) + bh_ref[...]
    obj_ref[0] = head[:_A, :]
    bb_ref[0] = head[_A:, :].T


def _make_anchors_const(batch, h, w):
    cx = (jnp.arange(w, dtype=jnp.float32) + 0.5) * _STRIDE
    cy = (jnp.arange(h, dtype=jnp.float32) + 0.5) * _STRIDE
    cyg, cxg = jnp.meshgrid(cy, cx, indexing='ij')
    whs = []
    for s in _SCALES:
        for r in _RATIOS:
            whs.append((s * np.sqrt(r), s / np.sqrt(r)))
    wh = jnp.asarray(np.array(whs, dtype=np.float32))
    cxg = jnp.broadcast_to(cxg[:, :, None], (h, w, _A))
    cyg = jnp.broadcast_to(cyg[:, :, None], (h, w, _A))
    aw = jnp.broadcast_to(wh[None, None, :, 0], (h, w, _A))
    ah = jnp.broadcast_to(wh[None, None, :, 1], (h, w, _A))
    anchors = jnp.stack([cxg, cyg, aw, ah], axis=-1).reshape(h * w * _A, 4)
    return jnp.broadcast_to(anchors[None], (batch, h * w * _A, 4))


def kernel(features, W_conv, b_conv, W_obj, b_obj, W_bbox, b_bbox):
    b, c, h, w = features.shape
    hw = h * w
    nhead = 5 * _A

    xf = features.reshape(b, c, hw)
    wt = jnp.transpose(W_conv, (2, 3, 0, 1)).reshape(9, c, c)
    wh_w = jnp.concatenate(
        [W_obj.reshape(_A, c), W_bbox.reshape(4 * _A, c)], axis=0)
    bh = jnp.concatenate([b_obj, b_bbox])[:, None]
    bc = b_conv[:, None]

    obj_out, bb_out = pl.pallas_call(
        functools.partial(_rpn_kernel, c=c, h=h, w=w),
        grid=(b,),
        in_specs=[
            pl.BlockSpec((1, c, hw), lambda i: (i, 0, 0)),
            pl.BlockSpec((9, c, c), lambda i: (0, 0, 0)),
            pl.BlockSpec((c, 1), lambda i: (0, 0)),
            pl.BlockSpec((nhead, c), lambda i: (0, 0)),
            pl.BlockSpec((nhead, 1), lambda i: (0, 0)),
        ],
        out_specs=[
            pl.BlockSpec((1, _A, hw), lambda i: (i, 0, 0)),
            pl.BlockSpec((1, hw, 4 * _A), lambda i: (i, 0, 0)),
        ],
        out_shape=[
            jax.ShapeDtypeStruct((b, _A, hw), jnp.float32),
            jax.ShapeDtypeStruct((b, hw, 4 * _A), jnp.float32),
        ],
    )(xf, wt, bc, wh_w, bh)

    objness = obj_out.reshape(b, _A * hw, 1)
    bb = bb_out.reshape(b, hw * _A, 4)
    anchors = _make_anchors_const(b, h, w)
    return (objness, bb, anchors)


# speedup vs baseline: 1.0681x; 1.0681x over previous
import functools
import numpy as np
import jax
import jax.numpy as jnp
from jax.experimental import pallas as pl

_A = 9
_STRIDE = 16
_SCALES = (64.0, 128.0, 256.0)
_RATIOS = (0.5, 1.0, 2.0)


def _rpn_kernel(x_ref, wt_ref, bc_ref, wh_ref, bh_ref, obj_ref, bb_ref,
                *, c, h, w):
    hw = h * w
    x = x_ref[0]

    pos = jax.lax.broadcasted_iota(jnp.int32, (1, hw), 1)
    col = pos % w
    ones = jnp.ones((1, hw), jnp.bfloat16)
    wmask = [ (col != 0).astype(jnp.bfloat16),
              ones,
              (col != w - 1).astype(jnp.bfloat16) ]
    hmask = [ (pos >= w).astype(jnp.bfloat16),
              ones,
              (pos < hw - w).astype(jnp.bfloat16) ]

    acc = jnp.zeros((c, hw), dtype=jnp.float32)
    for k in range(9):
        i, j = k // 3, k % 3
        s = (i - 1) * w + (j - 1)
        xs = x if s == 0 else jnp.roll(x, -s, axis=1)
        m = None
        if i != 1:
            m = hmask[i]
        if j != 1:
            m = wmask[j] if m is None else m * wmask[j]
        xs = xs if m is None else xs * m
        acc = acc + jnp.dot(wt_ref[k], xs, preferred_element_type=jnp.float32)
    y = jnp.maximum(acc + bc_ref[...], 0.0).astype(jnp.bfloat16)
    head = jnp.dot(wh_ref[...], y, preferred_element_type=jnp.float32) + bh_ref[...]
    obj_ref[0] = head[:_A, :]
    bb_ref[0] = head[_A:, :].T


def _make_anchors_const(batch, h, w):
    cx = (jnp.arange(w, dtype=jnp.float32) + 0.5) * _STRIDE
    cy = (jnp.arange(h, dtype=jnp.float32) + 0.5) * _STRIDE
    cyg, cxg = jnp.meshgrid(cy, cx, indexing='ij')
    whs = []
    for s in _SCALES:
        for r in _RATIOS:
            whs.append((s * np.sqrt(r), s / np.sqrt(r)))
    wh = jnp.asarray(np.array(whs, dtype=np.float32))
    cxg = jnp.broadcast_to(cxg[:, :, None], (h, w, _A))
    cyg = jnp.broadcast_to(cyg[:, :, None], (h, w, _A))
    aw = jnp.broadcast_to(wh[None, None, :, 0], (h, w, _A))
    ah = jnp.broadcast_to(wh[None, None, :, 1], (h, w, _A))
    anchors = jnp.stack([cxg, cyg, aw, ah], axis=-1).reshape(h * w * _A, 4)
    return jnp.broadcast_to(anchors[None], (batch, h * w * _A, 4))


def kernel(features, W_conv, b_conv, W_obj, b_obj, W_bbox, b_bbox):
    b, c, h, w = features.shape
    hw = h * w
    nhead = 5 * _A

    xf = features.reshape(b, c, hw).astype(jnp.bfloat16)
    wt = jnp.transpose(W_conv, (2, 3, 0, 1)).reshape(9, c, c).astype(jnp.bfloat16)
    wh_w = jnp.concatenate(
        [W_obj.reshape(_A, c), W_bbox.reshape(4 * _A, c)], axis=0).astype(jnp.bfloat16)
    bh = jnp.concatenate([b_obj, b_bbox])[:, None]
    bc = b_conv[:, None]

    obj_out, bb_out = pl.pallas_call(
        functools.partial(_rpn_kernel, c=c, h=h, w=w),
        grid=(b,),
        in_specs=[
            pl.BlockSpec((1, c, hw), lambda i: (i, 0, 0)),
            pl.BlockSpec((9, c, c), lambda i: (0, 0, 0)),
            pl.BlockSpec((c, 1), lambda i: (0, 0)),
            pl.BlockSpec((nhead, c), lambda i: (0, 0)),
            pl.BlockSpec((nhead, 1), lambda i: (0, 0)),
        ],
        out_specs=[
            pl.BlockSpec((1, _A, hw), lambda i: (i, 0, 0)),
            pl.BlockSpec((1, hw, 4 * _A), lambda i: (i, 0, 0)),
        ],
        out_shape=[
            jax.ShapeDtypeStruct((b, _A, hw), jnp.float32),
            jax.ShapeDtypeStruct((b, hw, 4 * _A), jnp.float32),
        ],
    )(xf, wt, bc, wh_w, bh)

    objness = obj_out.reshape(b, _A * hw, 1)
    bb = bb_out.reshape(b, hw * _A, 4)
    anchors = _make_anchors_const(b, h, w)
    return (objness, bb, anchors)
